# Initial kernel scaffold; baseline (speedup 1.0000x reference)
import jax
import jax.numpy as jnp
from jax.experimental import pallas as pl


def kernel(state, w1, b1, w2, b2, w3, b3, w4, b4, w5, b5):
    raise NotImplementedError("write your pallas kernel here")


# speedup vs baseline: 3.1851x; 3.1851x over previous
import functools

import jax
import jax.numpy as jnp
from jax.experimental import pallas as pl
from jax.experimental.pallas import tpu as pltpu


def _round_up(x, m):
    return (x + m - 1) // m * m


def _fused_kernel(x_ref, w1_ref, b1_ref, w2_ref, b2_ref, w3_ref, b3_ref,
                  w4_ref, b4_ref, w5_ref, b5_ref, out_ref, *, tb):
    w1v = w1_ref[...]
    b1v = b1_ref[...]
    w2v = w2_ref[...]
    b2v = b2_ref[...]
    w3v = w3_ref[...]
    w4v = w4_ref[...]
    b4v = b4_ref[...]
    w5v = w5_ref[...]
    b5v = b5_ref[...]

    pooled = []
    for b in range(tb):
        xt = x_ref[b].astype(jnp.bfloat16)
        h = jax.lax.dot_general(
            xt, w1v, (((0,), (0,)), ((), ())),
            preferred_element_type=jnp.float32)
        h = jnp.maximum(h + b1v, 0.0).astype(jnp.bfloat16)
        h = jnp.dot(h, w2v, preferred_element_type=jnp.float32) + b2v
        h = jnp.maximum(h, 0.0).astype(jnp.bfloat16)
        h = jnp.dot(h, w3v, preferred_element_type=jnp.float32)
        pooled.append(jnp.max(h, axis=0, keepdims=True))
    g = pooled[0] if tb == 1 else jnp.concatenate(pooled, axis=0)
    g = jnp.maximum(g + b3_ref[...], 0.0).astype(jnp.bfloat16)

    y = jnp.dot(g, w4v, preferred_element_type=jnp.float32) + b4v
    y = jnp.maximum(y, 0.0).astype(jnp.bfloat16)
    y = jnp.dot(y, w5v, preferred_element_type=jnp.float32) + b5v
    out_ref[...] = jnp.maximum(y, 0.0)


def kernel(state, w1, b1, w2, b2, w3, b3, w4, b4, w5, b5):
    B, C, N = state.shape
    F = w1.shape[1]
    F2 = w2.shape[1]
    H = w5.shape[1]
    F_p = _round_up(F, 128)
    F2_p = _round_up(F2, 128)
    H_p = _round_up(H, 128)

    for t in (16, 8, 4, 2, 1):
        if B % t == 0:
            TB = t
            break

    def pad2(a, rows, cols):
        return jnp.pad(a, ((0, rows - a.shape[0]), (0, cols - a.shape[1])))

    cdt = jnp.bfloat16
    w1p = pad2(w1, C, F_p).astype(cdt)
    w2p = pad2(w2, F_p, F2_p).astype(cdt)
    w3p = pad2(w3, F2_p, H_p).astype(cdt)
    w4p = pad2(w4, H_p, H_p).astype(cdt)
    w5p = pad2(w5, H_p, H_p).astype(cdt)
    b1p = pad2(b1, 1, F_p)
    b2p = pad2(b2, 1, F2_p)
    b3p = pad2(b3, 1, H_p)
    b4p = pad2(b4, 1, H_p)
    b5p = pad2(b5, 1, H_p)

    def const_spec(shape):
        return pl.BlockSpec(shape, lambda i, _nd=len(shape): (0,) * _nd)

    out = pl.pallas_call(
        functools.partial(_fused_kernel, tb=TB),
        out_shape=jax.ShapeDtypeStruct((B, H_p), jnp.float32),
        grid_spec=pltpu.PrefetchScalarGridSpec(
            num_scalar_prefetch=0,
            grid=(B // TB,),
            in_specs=[
                pl.BlockSpec((TB, C, N), lambda i: (i, 0, 0)),
                const_spec((C, F_p)), const_spec((1, F_p)),
                const_spec((F_p, F2_p)), const_spec((1, F2_p)),
                const_spec((F2_p, H_p)), const_spec((1, H_p)),
                const_spec((H_p, H_p)), const_spec((1, H_p)),
                const_spec((H_p, H_p)), const_spec((1, H_p)),
            ],
            out_specs=pl.BlockSpec((TB, H_p), lambda i: (i, 0)),
        ),
        compiler_params=pltpu.CompilerParams(
            dimension_semantics=("parallel",),
            vmem_limit_bytes=64 * 1024 * 1024,
        ),
    )(state, w1p, b1p, w2p, b2p, w3p, b3p, w4p, b4p, w5p, b5p)

    return out[:, :H] if H_p != H else out


# speedup vs baseline: 4.1286x; 1.2962x over previous
import functools

import jax
import jax.numpy as jnp
import numpy as np
from jax.experimental import pallas as pl
from jax.experimental.pallas import tpu as pltpu
from jax.sharding import PartitionSpec as P


def _round_up(x, m):
    return (x + m - 1) // m * m


def _tree_max(h):
    r = h.shape[0]
    while r > 8:
        r //= 2
        h = jnp.maximum(h[:r], h[r:2 * r])
    return jnp.max(h, axis=0, keepdims=True)


def _fused_kernel(x_ref, w1t_ref, w2t_ref, b2_ref, w3_ref, b3_ref,
                  w4_ref, b4_ref, w5_ref, b5_ref, out_ref, *, tb):
    w1tv = w1t_ref[...]
    w2tv = w2t_ref[...]
    b2v = b2_ref[...]
    w3v = w3_ref[...]
    w4v = w4_ref[...]
    b4v = b4_ref[...]
    w5v = w5_ref[...]
    b5v = b5_ref[...]

    n = x_ref.shape[2]
    ones_row = jnp.ones((1, n), jnp.bfloat16)
    h2s = []
    for b in range(tb):
        xt4 = jnp.concatenate(
            [x_ref[b].astype(jnp.bfloat16), ones_row], axis=0)
        h = jnp.dot(w1tv, xt4, preferred_element_type=jnp.float32)
        h = jnp.maximum(h, 0.0).astype(jnp.bfloat16)
        h = jnp.dot(w2tv, h, preferred_element_type=jnp.float32) + b2v
        h2s.append(jnp.maximum(h, 0.0).astype(jnp.bfloat16))
    pooled = []
    for b in range(tb):
        h = jax.lax.dot_general(
            h2s[b], w3v, (((0,), (0,)), ((), ())),
            preferred_element_type=jnp.float32)
        pooled.append(_tree_max(h))
    g = pooled[0] if tb == 1 else jnp.concatenate(pooled, axis=0)
    g = jnp.maximum(g + b3_ref[...], 0.0).astype(jnp.bfloat16)

    y = jnp.dot(g, w4v, preferred_element_type=jnp.float32) + b4v
    y = jnp.maximum(y, 0.0).astype(jnp.bfloat16)
    y = jnp.dot(y, w5v, preferred_element_type=jnp.float32) + b5v
    out_ref[...] = jnp.maximum(y, 0.0)


def kernel(state, w1, b1, w2, b2, w3, b3, w4, b4, w5, b5):
    B, C, N = state.shape
    F = w1.shape[1]
    F2 = w2.shape[1]
    H = w5.shape[1]
    F_p = _round_up(F, 128)
    F2_p = _round_up(F2, 128)
    H_p = _round_up(H, 128)

    for t in (16, 8, 4, 2, 1):
        if B % t == 0:
            TB = t
            break

    def pad2(a, rows, cols):
        return jnp.pad(a, ((0, rows - a.shape[0]), (0, cols - a.shape[1])))

    cdt = jnp.bfloat16
    w1tp = pad2(jnp.concatenate([w1, b1], axis=0).T, F_p, C + 1).astype(cdt)
    w2tp = pad2(w2.T, F2_p, F_p).astype(cdt)
    w3p = pad2(w3, F2_p, H_p).astype(cdt)
    w4p = pad2(w4, H_p, H_p).astype(cdt)
    w5p = pad2(w5, H_p, H_p).astype(cdt)
    b2p = pad2(b2.T, F2_p, 1)
    b3p = pad2(b3, 1, H_p)
    b4p = pad2(b4, 1, H_p)
    b5p = pad2(b5, 1, H_p)

    def const_spec(shape):
        return pl.BlockSpec(shape, lambda i, _nd=len(shape): (0,) * _nd)

    def call_pallas(x_local, *ws):
        b_local = x_local.shape[0]
        return pl.pallas_call(
            functools.partial(_fused_kernel, tb=TB),
            out_shape=jax.ShapeDtypeStruct((b_local, H_p), jnp.float32),
            grid_spec=pltpu.PrefetchScalarGridSpec(
                num_scalar_prefetch=0,
                grid=(b_local // TB,),
                in_specs=[
                    pl.BlockSpec((TB, C, N), lambda i: (i, 0, 0)),
                    const_spec((F_p, C + 1)),
                    const_spec((F2_p, F_p)), const_spec((F2_p, 1)),
                    const_spec((F2_p, H_p)), const_spec((1, H_p)),
                    const_spec((H_p, H_p)), const_spec((1, H_p)),
                    const_spec((H_p, H_p)), const_spec((1, H_p)),
                ],
                out_specs=pl.BlockSpec((TB, H_p), lambda i: (i, 0)),
            ),
            compiler_params=pltpu.CompilerParams(
                dimension_semantics=("parallel",),
                vmem_limit_bytes=64 * 1024 * 1024,
            ),
        )(x_local, *ws)

    xb = state.astype(jnp.bfloat16)
    weights = (w1tp, w2tp, b2p, w3p, b3p, w4p, b4p, w5p, b5p)

    devs = jax.devices()
    nd = len(devs)
    if nd > 1 and B % (nd * TB) == 0:
        mesh = jax.sharding.Mesh(np.array(devs), ("x",))
        out = jax.shard_map(
            call_pallas, mesh=mesh,
            in_specs=(P("x", None, None),) + (P(None, None),) * len(weights),
            out_specs=P("x", None), check_vma=False,
        )(xb, *weights)
    else:
        out = call_pallas(xb, *weights)

    return out[:, :H] if H_p != H else out


# speedup vs baseline: 4.2630x; 1.0326x over previous
import functools

import jax
import jax.numpy as jnp
import numpy as np
from jax.experimental import pallas as pl
from jax.experimental.pallas import tpu as pltpu
from jax.sharding import PartitionSpec as P


def _round_up(x, m):
    return (x + m - 1) // m * m


def _tree_max(h):
    r = h.shape[0]
    while r > 8:
        r //= 2
        h = jnp.maximum(h[:r], h[r:2 * r])
    return jnp.max(h, axis=0, keepdims=True)


def _fused_kernel(x_ref, w1t_ref, w2t_ref, b2_ref, w3_ref, b3_ref,
                  w4_ref, b4_ref, w5_ref, b5_ref, out_ref, *, tb):
    w1tv = w1t_ref[...]
    w2tv = w2t_ref[...]
    b2v = b2_ref[...]
    w3v = w3_ref[...]
    w4v = w4_ref[...]
    b4v = b4_ref[...]
    w5v = w5_ref[...]
    b5v = b5_ref[...]

    n = x_ref.shape[2]
    ones_row = jnp.ones((1, n), jnp.bfloat16)
    h2s = []
    for b in range(tb):
        xt4 = jnp.concatenate(
            [x_ref[b].astype(jnp.bfloat16), ones_row], axis=0)
        h = jnp.dot(w1tv, xt4, preferred_element_type=jnp.float32)
        h = jnp.maximum(h, 0.0).astype(jnp.bfloat16)
        h = jnp.dot(w2tv, h, preferred_element_type=jnp.float32) + b2v
        h2s.append(jnp.maximum(h, 0.0).astype(jnp.bfloat16))
    pooled = []
    for b in range(tb):
        h = jax.lax.dot_general(
            h2s[b], w3v, (((0,), (0,)), ((), ())),
            preferred_element_type=jnp.float32)
        pooled.append(_tree_max(h))
    g = pooled[0] if tb == 1 else jnp.concatenate(pooled, axis=0)
    g = jnp.maximum(g + b3_ref[...], 0.0).astype(jnp.bfloat16)

    y = jnp.dot(g, w4v, preferred_element_type=jnp.float32) + b4v
    y = jnp.maximum(y, 0.0).astype(jnp.bfloat16)
    y = jnp.dot(y, w5v, preferred_element_type=jnp.float32) + b5v
    out_ref[...] = jnp.maximum(y, 0.0)


def kernel(state, w1, b1, w2, b2, w3, b3, w4, b4, w5, b5):
    B, C, N = state.shape
    F = w1.shape[1]
    F2 = w2.shape[1]
    H = w5.shape[1]
    F_p = _round_up(F, 128)
    F2_p = _round_up(F2, 128)
    H_p = _round_up(H, 128)

    for t in (32, 16, 8, 4, 2, 1):
        if B % t == 0:
            TB = t
            break

    def pad2(a, rows, cols):
        return jnp.pad(a, ((0, rows - a.shape[0]), (0, cols - a.shape[1])))

    cdt = jnp.bfloat16
    w1tp = pad2(jnp.concatenate([w1, b1], axis=0).T, F_p, C + 1).astype(cdt)
    w2tp = pad2(w2.T, F2_p, F_p).astype(cdt)
    w3p = pad2(w3, F2_p, H_p).astype(cdt)
    w4p = pad2(w4, H_p, H_p).astype(cdt)
    w5p = pad2(w5, H_p, H_p).astype(cdt)
    b2p = pad2(b2.T, F2_p, 1)
    b3p = pad2(b3, 1, H_p)
    b4p = pad2(b4, 1, H_p)
    b5p = pad2(b5, 1, H_p)

    def const_spec(shape):
        return pl.BlockSpec(shape, lambda i, _nd=len(shape): (0,) * _nd)

    def call_pallas(x_local, *ws):
        b_local = x_local.shape[0]
        return pl.pallas_call(
            functools.partial(_fused_kernel, tb=TB),
            out_shape=jax.ShapeDtypeStruct((b_local, H_p), jnp.float32),
            grid_spec=pltpu.PrefetchScalarGridSpec(
                num_scalar_prefetch=0,
                grid=(b_local // TB,),
                in_specs=[
                    pl.BlockSpec((TB, C, N), lambda i: (i, 0, 0)),
                    const_spec((F_p, C + 1)),
                    const_spec((F2_p, F_p)), const_spec((F2_p, 1)),
                    const_spec((F2_p, H_p)), const_spec((1, H_p)),
                    const_spec((H_p, H_p)), const_spec((1, H_p)),
                    const_spec((H_p, H_p)), const_spec((1, H_p)),
                ],
                out_specs=pl.BlockSpec((TB, H_p), lambda i: (i, 0)),
            ),
            compiler_params=pltpu.CompilerParams(
                dimension_semantics=("parallel",),
                vmem_limit_bytes=64 * 1024 * 1024,
            ),
        )(x_local, *ws)

    xb = state.astype(jnp.bfloat16)
    weights = (w1tp, w2tp, b2p, w3p, b3p, w4p, b4p, w5p, b5p)

    devs = jax.devices()
    nd = len(devs)
    if nd > 1 and B % (nd * TB) == 0:
        mesh = jax.sharding.Mesh(np.array(devs), ("x",))
        out = jax.shard_map(
            call_pallas, mesh=mesh,
            in_specs=(P("x", None, None),) + (P(None, None),) * len(weights),
            out_specs=P("x", None), check_vma=False,
        )(xb, *weights)
    else:
        out = call_pallas(xb, *weights)

    return out[:, :H] if H_p != H else out
